# baseline (device time: 35243 ns/iter reference)
import jax
import jax.numpy as jnp
from jax import lax
from jax.experimental import pallas as pl
from jax.experimental.pallas import tpu as pltpu

_sem_signal = getattr(pl, "semaphore_signal", None) or pltpu.semaphore_signal
_sem_wait = getattr(pl, "semaphore_wait", None) or pltpu.semaphore_wait
_DeviceIdType = getattr(pl, "DeviceIdType", None) or pltpu.DeviceIdType
_CompilerParams = getattr(pltpu, "CompilerParams", None) or getattr(
    pltpu, "TPUCompilerParams"
)

M = 1024
D = 1024
EPS = 1e-6
NB = 4
BM = M // NB
C = 4
CM = BM // C


def kernel(partial, resid, gamma):
    p = partial.reshape(M, D)
    g = gamma.reshape(1, D)

    def body(
        p_ref, r_ref, g_ref, out_ref, comm_x,
        sx_s, sx_r, yd_s, yd_r, zd_s, zd_r,
    ):
        my_x = lax.axis_index("x")
        my_y = lax.axis_index("y")
        my_z = lax.axis_index("z")
        xn = (1 - my_x, my_y, my_z)
        yn = (my_x, 1 - my_y, my_z)
        zn = (my_x, my_y, 1 - my_z)

        b = 2 * my_y + my_z
        bd = 3 - b

        def rows(blk, c):
            return pl.ds(blk * BM + c * CM, CM)

        barrier_sem = pltpu.get_barrier_semaphore()
        for nbr in (xn, yn, zn):
            _sem_signal(
                barrier_sem, inc=1, device_id=nbr,
                device_id_type=_DeviceIdType.MESH,
            )
        _sem_wait(barrier_sem, 3)

        x_rdma = []
        for i in range(2 * C):
            blk, c = (b, i) if i < C else (bd, i - C)
            r = pltpu.make_async_remote_copy(
                src_ref=p_ref.at[rows(blk, c), :],
                dst_ref=comm_x.at[pl.ds(i * CM, CM), :],
                send_sem=sx_s.at[i],
                recv_sem=sx_r.at[i],
                device_id=xn,
                device_id_type=_DeviceIdType.MESH,
            )
            r.start()
            x_rdma.append(r)

        def compute(blk, c, slot):
            y = p_ref[rows(blk, c), :] + comm_x[pl.ds(slot * CM, CM), :] \
                + r_ref[rows(blk, c), :]
            ms = jnp.mean(y * y, axis=-1, keepdims=True)
            out_ref[rows(blk, c), :] = y * lax.rsqrt(ms + EPS) * g_ref[...]

        yd, zd = [], []
        for c in range(C):
            x_rdma[c].wait_recv()
            compute(b, c, c)
            for sems_s, sems_r, nbr, acc in (
                (yd_s, yd_r, yn, yd),
                (zd_s, zd_r, zn, zd),
            ):
                r = pltpu.make_async_remote_copy(
                    src_ref=out_ref.at[rows(b, c), :],
                    dst_ref=out_ref.at[rows(b, c), :],
                    send_sem=sems_s.at[c],
                    recv_sem=sems_r.at[c],
                    device_id=nbr,
                    device_id_type=_DeviceIdType.MESH,
                )
                r.start()
                acc.append(r)

        for c in range(C):
            x_rdma[C + c].wait_recv()
            compute(bd, c, C + c)

        for r in yd + zd:
            r.wait_recv()
        for r in x_rdma + yd + zd:
            r.wait_send()

    return pl.pallas_call(
        body,
        out_shape=jax.ShapeDtypeStruct((M, D), jnp.float32),
        in_specs=[
            pl.BlockSpec(memory_space=pltpu.VMEM),
            pl.BlockSpec(memory_space=pltpu.VMEM),
            pl.BlockSpec(memory_space=pltpu.VMEM),
        ],
        out_specs=pl.BlockSpec(memory_space=pltpu.VMEM),
        scratch_shapes=[
            pltpu.VMEM((2 * BM, D), jnp.float32),
            pltpu.SemaphoreType.DMA((2 * C,)),
            pltpu.SemaphoreType.DMA((2 * C,)),
            pltpu.SemaphoreType.DMA((C,)),
            pltpu.SemaphoreType.DMA((C,)),
            pltpu.SemaphoreType.DMA((C,)),
            pltpu.SemaphoreType.DMA((C,)),
        ],
        compiler_params=_CompilerParams(collective_id=0),
    )(p, resid, g)


# device time: 35141 ns/iter; 1.0029x vs baseline; 1.0029x over previous
import jax
import jax.numpy as jnp
from jax import lax
from jax.experimental import pallas as pl
from jax.experimental.pallas import tpu as pltpu

_sem_signal = getattr(pl, "semaphore_signal", None) or pltpu.semaphore_signal
_sem_wait = getattr(pl, "semaphore_wait", None) or pltpu.semaphore_wait
_DeviceIdType = getattr(pl, "DeviceIdType", None) or pltpu.DeviceIdType
_CompilerParams = getattr(pltpu, "CompilerParams", None) or getattr(
    pltpu, "TPUCompilerParams"
)

M = 1024
D = 1024
EPS = 1e-6
NB = 4
BM = M // NB
C = 4
CM = BM // C


def kernel(partial, resid, gamma):
    def body(
        p_ref, r_ref, g_ref, out_ref, comm_x,
        sx_s, sx_r, yd_s, yd_r, zd_s, zd_r,
    ):
        my_x = lax.axis_index("x")
        my_y = lax.axis_index("y")
        my_z = lax.axis_index("z")
        xn = (1 - my_x, my_y, my_z)
        yn = (my_x, 1 - my_y, my_z)
        zn = (my_x, my_y, 1 - my_z)

        b = 2 * my_y + my_z
        bd = 3 - b

        def rows(blk, c):
            return pl.ds(blk * BM + c * CM, CM)

        barrier_sem = pltpu.get_barrier_semaphore()
        for nbr in (xn, yn, zn):
            _sem_signal(
                barrier_sem, inc=1, device_id=nbr,
                device_id_type=_DeviceIdType.MESH,
            )
        _sem_wait(barrier_sem, 3)

        x_rdma = []
        for i in range(2 * C):
            blk, c = (b, i) if i < C else (bd, i - C)
            r = pltpu.make_async_remote_copy(
                src_ref=p_ref.at[0, rows(blk, c), :],
                dst_ref=comm_x.at[pl.ds(i * CM, CM), :],
                send_sem=sx_s.at[i],
                recv_sem=sx_r.at[i],
                device_id=xn,
                device_id_type=_DeviceIdType.MESH,
            )
            r.start()
            x_rdma.append(r)

        def compute(blk, c, slot):
            y = p_ref[0, rows(blk, c), :] + comm_x[pl.ds(slot * CM, CM), :] \
                + r_ref[rows(blk, c), :]
            ms = jnp.mean(y * y, axis=-1, keepdims=True)
            out_ref[rows(blk, c), :] = (
                y * lax.rsqrt(ms + EPS) * g_ref[...].reshape(1, D)
            )

        yd, zd = [], []
        for c in range(C):
            x_rdma[c].wait_recv()
            compute(b, c, c)
            for sems_s, sems_r, nbr, acc in (
                (yd_s, yd_r, yn, yd),
                (zd_s, zd_r, zn, zd),
            ):
                r = pltpu.make_async_remote_copy(
                    src_ref=out_ref.at[rows(b, c), :],
                    dst_ref=out_ref.at[rows(b, c), :],
                    send_sem=sems_s.at[c],
                    recv_sem=sems_r.at[c],
                    device_id=nbr,
                    device_id_type=_DeviceIdType.MESH,
                )
                r.start()
                acc.append(r)

        for c in range(C):
            x_rdma[C + c].wait_recv()
            compute(bd, c, C + c)

        for r in yd + zd:
            r.wait_recv()
        for r in x_rdma + yd + zd:
            r.wait_send()

    return pl.pallas_call(
        body,
        out_shape=jax.ShapeDtypeStruct((M, D), jnp.float32),
        in_specs=[
            pl.BlockSpec(memory_space=pltpu.VMEM),
            pl.BlockSpec(memory_space=pltpu.VMEM),
            pl.BlockSpec(memory_space=pltpu.VMEM),
        ],
        out_specs=pl.BlockSpec(memory_space=pltpu.VMEM),
        scratch_shapes=[
            pltpu.VMEM((2 * BM, D), jnp.float32),
            pltpu.SemaphoreType.DMA((2 * C,)),
            pltpu.SemaphoreType.DMA((2 * C,)),
            pltpu.SemaphoreType.DMA((C,)),
            pltpu.SemaphoreType.DMA((C,)),
            pltpu.SemaphoreType.DMA((C,)),
            pltpu.SemaphoreType.DMA((C,)),
        ],
        compiler_params=_CompilerParams(collective_id=0),
    )(partial, resid, gamma)
